# baseline (speedup 1.0000x reference)
import functools

import jax
import jax.numpy as jnp
from jax import lax
from jax.experimental import pallas as pl
from jax.experimental.pallas import tpu as pltpu
from jax.experimental.pallas import tpu_sc as plsc

_H = 512
_W = 512
_STUFF = 53
_NI = 64
_M = 100
_COUT = _STUFF + _NI
_B = 256
_BAND = 384
_NS = 8
_LEAD = 4


def _stripe_copy(chan_ref, par_ref, sem_ref, stripe_ref, stripe_sem, n):
    yb = pl.multiple_of(par_ref[6, n], 128)
    xs = pl.multiple_of(par_ref[7, n], 128)
    return pltpu.make_async_copy(
        sem_ref.at[chan_ref[n], pl.ds(yb, _B), pl.ds(xs, _B)],
        stripe_ref.at[n], stripe_sem.at[n])


def _sread(sem_ref, sbuf, rsem, n):
    return pltpu.make_async_copy(sem_ref.at[n], sbuf.at[lax.rem(n, _NS)],
                                 rsem.at[lax.rem(n, _NS)])


def _swrite(out_ref, sbuf, wsem, n):
    return pltpu.make_async_copy(sbuf.at[lax.rem(n, _NS)], out_ref.at[n],
                                 wsem.at[lax.rem(n, _NS)])


def _body(chan_ref, par_ref, sem_ref, mask_ref, out_ref, win_ref,
          stripe_ref, sbuf, mbuf, rsem, wsem,
          stripe_sem, msem):
    n = pl.program_id(0)

    @pl.when(n == 0)
    def _init():
        pltpu.make_async_copy(mask_ref, mbuf, msem).start()
        for i in range(_LEAD):
            _stripe_copy(chan_ref, par_ref, sem_ref, stripe_ref,
                         stripe_sem, i).start()

    @pl.when(n < _NI - _LEAD)
    def _stripe_prefetch():
        _stripe_copy(chan_ref, par_ref, sem_ref, stripe_ref,
                     stripe_sem, n + _LEAD).start()

    @pl.when((n >= _NS) & (n < _STUFF + _NS))
    def _stuff_wait_write():
        _swrite(out_ref, sbuf, wsem, n - _NS).wait()

    @pl.when(n < _STUFF)
    def _stuff_read():
        _sread(sem_ref, sbuf, rsem, n).start()

    @pl.when((n >= 2) & (n < _STUFF + 2))
    def _stuff_write():
        _sread(sem_ref, sbuf, rsem, n - 2).wait()
        _swrite(out_ref, sbuf, wsem, n - 2).start()

    @pl.when(n == 0)
    def _mask_wait():
        pltpu.make_async_copy(mask_ref, mbuf, msem).wait()

    by0 = par_ref[0, n]
    bx0 = par_ref[1, n]
    by1 = par_ref[2, n]
    bx1 = par_ref[3, n]
    cy2 = par_ref[4, n]
    cx2 = par_ref[5, n]
    yb = pl.multiple_of(par_ref[6, n], 128)
    xs = pl.multiple_of(par_ref[7, n], 128)
    yb3 = pl.multiple_of(par_ref[8, n], 128)

    bhf = (by1 - by0 + 1).astype(jnp.float32)
    bwf = (bx1 - bx0 + 1).astype(jnp.float32)

    def weights(base, x0, sizef, hi):
        ri = base + lax.broadcasted_iota(jnp.int32, (_B, 1), 0)
        rf = ri.astype(jnp.float32)
        s = (rf - x0.astype(jnp.float32) + 0.5) * (_M / sizef) - 0.5
        s = jnp.clip(s, 0.0, _M - 1.0)
        sf = jnp.floor(s)
        w = s - sf
        keep = (ri >= x0) & (ri <= hi)
        i0 = sf.astype(jnp.int32)
        i1 = jnp.minimum(i0 + 1, _M - 1)
        kk = lax.broadcasted_iota(jnp.int32, (_B, _M), 1)
        wm = (jnp.where(kk == i0, 1.0 - w, 0.0)
              + jnp.where(kk == i1, w, 0.0))
        return jnp.where(keep, wm, 0.0)

    wy = weights(yb, by0, bhf, by1)
    wx = weights(xs, bx0, bwf, bx1)
    m2d = mbuf[n]
    tmp = lax.dot_general(wy, m2d, (((1,), (0,)), ((), ())),
                          precision=lax.Precision.HIGHEST,
                          preferred_element_type=jnp.float32)
    val = lax.dot_general(tmp, wx, (((1,), (1,)), ((), ())),
                          precision=lax.Precision.HIGHEST,
                          preferred_element_type=jnp.float32)

    iy = yb + lax.broadcasted_iota(jnp.int32, (_B, 1), 0)
    ix = xs + lax.broadcasted_iota(jnp.int32, (1, _B), 1)
    rowm = ((iy >= by0) & (iy < cy2)).astype(jnp.float32)
    colm = ((ix >= bx0) & (ix < cx2)).astype(jnp.float32)

    _stripe_copy(chan_ref, par_ref, sem_ref, stripe_ref,
                 stripe_sem, n).wait()
    win_ref[...] = (val + stripe_ref[n] * (rowm * colm))[None]


_SC_MESH = plsc.VectorSubcoreMesh(core_axis_name="c", subcore_axis_name="s")


def _sc_extract(parv, row, n):
    lanes = lax.broadcasted_iota(jnp.int32, (16,), 0)
    acc = jnp.int32(0)
    for j in range(_NI // 16):
        v = parv[row, pl.ds(j * 16, 16)]
        acc = acc + jnp.sum(jnp.where(lanes + j * 16 == n, v, 0))
    return acc


@functools.partial(
    pl.kernel,
    mesh=_SC_MESH,
    scratch_types=[
        pltpu.VMEM((2, _NI + 16), jnp.int32),
        pltpu.VMEM((_B, _B), jnp.float32),
        pltpu.VMEM_SHARED((128, _W), jnp.float32),
        pltpu.SemaphoreType.DMA,
    ],
)
def _sc_fill(buf_ref, win_ref, par_ref, zeros_ref,
             parv, wbuf, zsh, dsem):
    cid = lax.axis_index("c")
    sid = lax.axis_index("s")
    wid = sid * 2 + cid

    pltpu.sync_copy(par_ref, parv)

    @pl.when(sid == 0)
    def _zinit():
        pltpu.make_async_copy(zeros_ref, zsh, dsem).start()
        pltpu.make_async_copy(zeros_ref, zsh, dsem).wait()

    plsc.subcore_barrier()

    for k in range(2):
        n = wid * 2 + k
        yb = parv[0, pl.ds(n, 16)][0] * 128
        xs = parv[1, pl.ds(n, 16)][0] * 128
        pltpu.make_async_copy(win_ref.at[n], wbuf, dsem).start()
        for j in range(4):
            pltpu.make_async_copy(
                zsh, buf_ref.at[_STUFF + n, pl.ds(j * 128, 128), :],
                dsem).start()
        for j in range(4):
            pltpu.make_async_copy(
                zsh, buf_ref.at[_STUFF + n, pl.ds(j * 128, 128), :],
                dsem).wait()
        pltpu.make_async_copy(win_ref.at[n], wbuf, dsem).wait()
        for r in range(2):
            pltpu.make_async_copy(
                wbuf.at[pl.ds(r * 128, 128), :],
                buf_ref.at[_STUFF + n, pl.ds(yb + r * 128, 128),
                           pl.ds(xs, _B)],
                dsem).start()
        for r in range(2):
            pltpu.make_async_copy(
                wbuf.at[pl.ds(r * 128, 128), :],
                buf_ref.at[_STUFF + n, pl.ds(yb + r * 128, 128),
                           pl.ds(xs, _B)],
                dsem).wait()


@jax.jit
def kernel(sem_seg_logits, mask_logits, boxes, cls_idx):
    sem = sem_seg_logits[0]
    masks = mask_logits[:, 0]

    bx0 = boxes[:, 0].astype(jnp.int32)
    by0 = boxes[:, 1].astype(jnp.int32)
    bx1 = boxes[:, 2].astype(jnp.int32)
    by1 = boxes[:, 3].astype(jnp.int32)
    cx2 = jnp.round(boxes[:, 2]).astype(jnp.int32) + 1
    cy2 = jnp.round(boxes[:, 3]).astype(jnp.int32) + 1
    yb = jnp.minimum((by0 // 128) * 128, _H - _B)
    xs = jnp.minimum((bx0 // 128) * 128, _W - _B)
    yb3 = jnp.where(by0 < 128, 0, 128)
    params = jnp.stack([by0, bx0, by1, bx1, cy2, cx2, yb, xs, yb3])

    chan_sel = _STUFF + cls_idx.astype(jnp.int32)

    grid_spec = pltpu.PrefetchScalarGridSpec(
        num_scalar_prefetch=2,
        grid=(_NI,),
        in_specs=[
            pl.BlockSpec(memory_space=pl.ANY),
            pl.BlockSpec(memory_space=pl.ANY),
        ],
        out_specs=[
            pl.BlockSpec(memory_space=pl.ANY),
            pl.BlockSpec((1, _B, _B),
                         lambda n, chan, par: (n, 0, 0)),
        ],
        scratch_shapes=[
            pltpu.VMEM((_NI, _B, _B), jnp.float32),
            pltpu.VMEM((_NS, _H, _W), jnp.float32),
            pltpu.VMEM((_NI, _M, _M), jnp.float32),
            pltpu.SemaphoreType.DMA((_NS,)),
            pltpu.SemaphoreType.DMA((_NS,)),
            pltpu.SemaphoreType.DMA((_NI,)),
            pltpu.SemaphoreType.DMA,
        ],
    )

    canvas, windows = pl.pallas_call(
        _body,
        grid_spec=grid_spec,
        out_shape=[jax.ShapeDtypeStruct((_COUT, _H, _W), jnp.float32),
                   jax.ShapeDtypeStruct((_NI, _B, _B), jnp.float32)],
        compiler_params=pltpu.CompilerParams(
            dimension_semantics=("arbitrary",),
        ),
    )(chan_sel, params, sem, masks)

    scpar = jnp.pad(jnp.stack([yb // 128, xs // 128]),
                    ((0, 0), (0, 16))).astype(jnp.int32)
    zeros_src = jnp.zeros((128, _W), jnp.float32)
    buf = jax.new_ref(canvas)
    _sc_fill(buf, windows, scpar, zeros_src)
    return buf[...][None]


# speedup vs baseline: 1.6193x; 1.6193x over previous
import jax
import jax.numpy as jnp
from jax import lax
from jax.experimental import pallas as pl
from jax.experimental.pallas import tpu as pltpu

_H = 512
_W = 512
_STUFF = 53
_NI = 64
_M = 100
_COUT = _STUFF + _NI
_B = 256
_BAND = 384
_NS = 8
_LEAD = 4


def _stripe_copy(chan_ref, par_ref, sem_ref, stripe_ref, stripe_sem, n):
    yb = pl.multiple_of(par_ref[6, n], 128)
    xs = pl.multiple_of(par_ref[7, n], 128)
    return pltpu.make_async_copy(
        sem_ref.at[chan_ref[n], pl.ds(yb, _B), pl.ds(xs, _B)],
        stripe_ref.at[n], stripe_sem.at[n])


def _band_copy(par_ref, out_ref, pbuf, band_sem, n):
    yb3 = pl.multiple_of(par_ref[8, n], 128)
    slot = lax.rem(n, 2)
    return pltpu.make_async_copy(
        pbuf.at[slot], out_ref.at[_STUFF + n, pl.ds(yb3, _BAND), :],
        band_sem.at[slot])


def _zero_copy(par_ref, out_ref, zbuf, zsem, n):
    yb3 = par_ref[8, n]
    jz = pl.multiple_of(jnp.where(yb3 == 0, _BAND, 0), 128)
    slot = lax.rem(n, 2)
    return pltpu.make_async_copy(
        zbuf, out_ref.at[_STUFF + n, pl.ds(jz, 128), :], zsem.at[slot])


def _sread(sem_ref, sbuf, rsem, n):
    return pltpu.make_async_copy(sem_ref.at[n], sbuf.at[lax.rem(n, _NS)],
                                 rsem.at[lax.rem(n, _NS)])


def _swrite(out_ref, sbuf, wsem, n):
    return pltpu.make_async_copy(sbuf.at[lax.rem(n, _NS)], out_ref.at[n],
                                 wsem.at[lax.rem(n, _NS)])


def _body(chan_ref, par_ref, sem_ref, mask_ref, out_ref,
          zbuf, pbuf, stripe_ref, sbuf, mbuf, rsem, wsem,
          stripe_sem, band_sem, zsem, msem):
    n = pl.program_id(0)

    @pl.when(n == 0)
    def _init():
        zbuf[...] = jnp.zeros((128, _W), jnp.float32)
        pbuf[...] = jnp.zeros((2, _BAND, _W), jnp.float32)
        pltpu.make_async_copy(mask_ref, mbuf, msem).start()
        for i in range(_LEAD):
            _stripe_copy(chan_ref, par_ref, sem_ref, stripe_ref,
                         stripe_sem, i).start()

    @pl.when(n < _NI - _LEAD)
    def _stripe_prefetch():
        _stripe_copy(chan_ref, par_ref, sem_ref, stripe_ref,
                     stripe_sem, n + _LEAD).start()

    @pl.when((n >= _NS) & (n < _STUFF + _NS))
    def _stuff_wait_write():
        _swrite(out_ref, sbuf, wsem, n - _NS).wait()

    @pl.when(n < _STUFF)
    def _stuff_read():
        _sread(sem_ref, sbuf, rsem, n).start()

    @pl.when((n >= 2) & (n < _STUFF + 2))
    def _stuff_write():
        _sread(sem_ref, sbuf, rsem, n - 2).wait()
        _swrite(out_ref, sbuf, wsem, n - 2).start()

    @pl.when(n == 0)
    def _mask_wait():
        pltpu.make_async_copy(mask_ref, mbuf, msem).wait()

    slot = lax.rem(n, 2)
    by0 = par_ref[0, n]
    bx0 = par_ref[1, n]
    by1 = par_ref[2, n]
    bx1 = par_ref[3, n]
    cy2 = par_ref[4, n]
    cx2 = par_ref[5, n]
    yb = pl.multiple_of(par_ref[6, n], 128)
    xs = pl.multiple_of(par_ref[7, n], 128)
    yb3 = pl.multiple_of(par_ref[8, n], 128)

    @pl.when(n >= 2)
    def _reclaim():
        _band_copy(par_ref, out_ref, pbuf, band_sem, n - 2).wait()
        _zero_copy(par_ref, out_ref, zbuf, zsem, n - 2).wait()
        yo = pl.multiple_of(par_ref[6, n - 2] - par_ref[8, n - 2], 128)
        xo = pl.multiple_of(par_ref[7, n - 2], 128)
        pbuf[slot, pl.ds(yo, _B), pl.ds(xo, _B)] = jnp.zeros(
            (_B, _B), jnp.float32)

    bhf = (by1 - by0 + 1).astype(jnp.float32)
    bwf = (bx1 - bx0 + 1).astype(jnp.float32)

    def weights(base, x0, sizef, hi):
        ri = base + lax.broadcasted_iota(jnp.int32, (_B, 1), 0)
        rf = ri.astype(jnp.float32)
        s = (rf - x0.astype(jnp.float32) + 0.5) * (_M / sizef) - 0.5
        s = jnp.clip(s, 0.0, _M - 1.0)
        sf = jnp.floor(s)
        w = s - sf
        keep = (ri >= x0) & (ri <= hi)
        i0 = sf.astype(jnp.int32)
        i1 = jnp.minimum(i0 + 1, _M - 1)
        kk = lax.broadcasted_iota(jnp.int32, (_B, _M), 1)
        wm = (jnp.where(kk == i0, 1.0 - w, 0.0)
              + jnp.where(kk == i1, w, 0.0))
        return jnp.where(keep, wm, 0.0)

    wy = weights(yb, by0, bhf, by1)
    wx = weights(xs, bx0, bwf, bx1)
    m2d = mbuf[n]
    tmp = lax.dot_general(wy, m2d, (((1,), (0,)), ((), ())),
                          precision=lax.Precision.HIGHEST,
                          preferred_element_type=jnp.float32)
    val = lax.dot_general(tmp, wx, (((1,), (1,)), ((), ())),
                          precision=lax.Precision.HIGHEST,
                          preferred_element_type=jnp.float32)

    iy = yb + lax.broadcasted_iota(jnp.int32, (_B, 1), 0)
    ix = xs + lax.broadcasted_iota(jnp.int32, (1, _B), 1)
    rowm = ((iy >= by0) & (iy < cy2)).astype(jnp.float32)
    colm = ((ix >= bx0) & (ix < cx2)).astype(jnp.float32)

    _stripe_copy(chan_ref, par_ref, sem_ref, stripe_ref,
                 stripe_sem, n).wait()
    res = val + stripe_ref[n] * (rowm * colm)
    pbuf[slot, pl.ds(yb - yb3, _B), pl.ds(xs, _B)] = res

    _band_copy(par_ref, out_ref, pbuf, band_sem, n).start()
    _zero_copy(par_ref, out_ref, zbuf, zsem, n).start()

    @pl.when(n == _NI - 1)
    def _drain():
        for m in (n - 1, n):
            _band_copy(par_ref, out_ref, pbuf, band_sem, m).wait()
            _zero_copy(par_ref, out_ref, zbuf, zsem, m).wait()


@jax.jit
def kernel(sem_seg_logits, mask_logits, boxes, cls_idx):
    sem = sem_seg_logits[0]
    masks = mask_logits[:, 0]

    bx0 = boxes[:, 0].astype(jnp.int32)
    by0 = boxes[:, 1].astype(jnp.int32)
    bx1 = boxes[:, 2].astype(jnp.int32)
    by1 = boxes[:, 3].astype(jnp.int32)
    cx2 = jnp.round(boxes[:, 2]).astype(jnp.int32) + 1
    cy2 = jnp.round(boxes[:, 3]).astype(jnp.int32) + 1
    yb = jnp.minimum((by0 // 128) * 128, _H - _B)
    xs = jnp.minimum((bx0 // 128) * 128, _W - _B)
    yb3 = jnp.where(by0 < 128, 0, 128)
    params = jnp.stack([by0, bx0, by1, bx1, cy2, cx2, yb, xs, yb3])

    chan_sel = _STUFF + cls_idx.astype(jnp.int32)

    grid_spec = pltpu.PrefetchScalarGridSpec(
        num_scalar_prefetch=2,
        grid=(_NI,),
        in_specs=[
            pl.BlockSpec(memory_space=pl.ANY),
            pl.BlockSpec(memory_space=pl.ANY),
        ],
        out_specs=pl.BlockSpec(memory_space=pl.ANY),
        scratch_shapes=[
            pltpu.VMEM((128, _W), jnp.float32),
            pltpu.VMEM((2, _BAND, _W), jnp.float32),
            pltpu.VMEM((_NI, _B, _B), jnp.float32),
            pltpu.VMEM((_NS, _H, _W), jnp.float32),
            pltpu.VMEM((_NI, _M, _M), jnp.float32),
            pltpu.SemaphoreType.DMA((_NS,)),
            pltpu.SemaphoreType.DMA((_NS,)),
            pltpu.SemaphoreType.DMA((_NI,)),
            pltpu.SemaphoreType.DMA((2,)),
            pltpu.SemaphoreType.DMA((2,)),
            pltpu.SemaphoreType.DMA,
        ],
    )

    out = pl.pallas_call(
        _body,
        grid_spec=grid_spec,
        out_shape=jax.ShapeDtypeStruct((_COUT, _H, _W), jnp.float32),
        compiler_params=pltpu.CompilerParams(
            dimension_semantics=("arbitrary",),
        ),
    )(chan_sel, params, sem, masks)
    return out[None]
